# Initial kernel scaffold; baseline (speedup 1.0000x reference)
import jax
import jax.numpy as jnp
from jax.experimental import pallas as pl


def kernel(queries, keys, k):
    raise NotImplementedError("write your pallas kernel here")


# speedup vs baseline: 1.1657x; 1.1657x over previous
import functools

import jax
import jax.numpy as jnp
from jax import lax
from jax.experimental import pallas as pl
from jax.experimental.pallas import tpu as pltpu

QB = 64
KB = 4096
TOPK = 10
RUN = 16


def _body(q_ref, kb_ref, d_ref, i_ref, p_ref, runv_ref, runi_ref, *,
          kg, k_total):
    ki = pl.program_id(1)

    q = q_ref[...]
    kb = kb_ref[...]
    dots = lax.dot_general(q, kb, (((1,), (1,)), ((), ())),
                           preferred_element_type=jnp.float32)
    q_sq = jnp.sum(q * q, axis=1, keepdims=True)
    ones = jnp.ones((1, 128), jnp.float32)
    k_sq = lax.dot_general(ones, kb * kb, (((1,), (1,)), ((), ())),
                           preferred_element_type=jnp.float32,
                           precision=lax.Precision.HIGHEST)
    dist = (q_sq + k_sq) - 2.0 * dots

    col = ki * KB + lax.broadcasted_iota(jnp.int32, (QB, KB), 1)
    dist = jnp.where(col < k_total, dist, jnp.inf)

    inf16 = jnp.full((QB, RUN), jnp.inf, jnp.float32)
    rv = jnp.where(ki == 0, inf16, runv_ref[...])
    ri = jnp.where(ki == 0, jnp.zeros((QB, RUN), jnp.int32), runi_ref[...])

    lane = lax.broadcasted_iota(jnp.int32, (QB, RUN), 1)
    big = jnp.int32(2**31 - 1)
    nv = inf16
    ni = jnp.zeros((QB, RUN), jnp.int32)
    for t in range(TOPK):
        m = jnp.minimum(jnp.min(dist, axis=1), jnp.min(rv, axis=1))
        mq = m[:, None]
        idxb = jnp.min(jnp.where(dist == mq, col, big), axis=1)
        idxr = jnp.min(jnp.where(rv == mq, ri, big), axis=1)
        sel = jnp.minimum(idxb, idxr)
        selq = sel[:, None]
        dist = jnp.where(col == selq, jnp.inf, dist)
        rv = jnp.where(ri == selq, jnp.inf, rv)
        nv = jnp.where(lane == t, mq, nv)
        ni = jnp.where(lane == t, selq, ni)

    runv_ref[...] = nv
    runi_ref[...] = ni

    @pl.when(ki == kg - 1)
    def _emit():
        d_ref[...] = nv
        i_ref[...] = ni
        valid = lane < TOPK
        mx = jnp.max(jnp.where(valid, nv, -jnp.inf), axis=1, keepdims=True)
        e = jnp.where(valid, jnp.exp(nv - mx), 0.0)
        p_ref[...] = e / jnp.sum(e, axis=1, keepdims=True)


def kernel(queries, keys, k):
    del k
    q_n, d = queries.shape
    k_n = keys.shape[0]
    kg = pl.cdiv(k_n, KB)
    kp = kg * KB
    keys_p = jnp.pad(keys, ((0, kp - k_n), (0, 0)))
    qg = q_n // QB

    grid = (qg, kg)
    out_shape = [
        jax.ShapeDtypeStruct((q_n, RUN), jnp.float32),
        jax.ShapeDtypeStruct((q_n, RUN), jnp.int32),
        jax.ShapeDtypeStruct((q_n, RUN), jnp.float32),
    ]
    out_specs = [pl.BlockSpec((QB, RUN), lambda qi, ki: (qi, 0))
                 for _ in range(3)]
    dd, ii, pp = pl.pallas_call(
        functools.partial(_body, kg=kg, k_total=k_n),
        grid=grid,
        in_specs=[
            pl.BlockSpec((QB, d), lambda qi, ki: (qi, 0)),
            pl.BlockSpec((KB, d), lambda qi, ki: (ki, 0)),
        ],
        out_specs=out_specs,
        out_shape=out_shape,
        scratch_shapes=[
            pltpu.VMEM((QB, RUN), jnp.float32),
            pltpu.VMEM((QB, RUN), jnp.int32),
        ],
    )(queries, keys_p)
    return (dd[:, :TOPK], ii[:, :TOPK], pp[:, :TOPK])
